# baseline (speedup 1.0000x reference)
import jax
import jax.numpy as jnp
from jax import lax
from jax.experimental import pallas as pl
from jax.experimental.pallas import tpu as pltpu
from jax.experimental.pallas import tpu_sc as plsc

B = 16384
S = 50
SP = 56
DIM = 32
DP = 128
NC = 2
NS = 16
NW = NC * NS
BR_PER_W = B // NW
CHUNK_BR = 8
NCHUNK = BR_PER_W // CHUNK_BR
NBUF = 3


def _gather_body(x_hbm, table_hbm, out_hbm,
                 idx_v, rows0, rows1, rows2,
                 gsem0, gsem1, gsem2, ssem0, ssem1, ssem2):
    rows = (rows0, rows1, rows2)
    gsem = (gsem0, gsem1, gsem2)
    ssem = (ssem0, ssem1, ssem2)

    wid = lax.axis_index("s") * NC + lax.axis_index("c")
    br_base = wid * BR_PER_W

    pltpu.sync_copy(x_hbm.at[pl.ds(br_base, BR_PER_W)], idx_v)

    def fire_gathers(c, b):
        for j in range(CHUNK_BR):
            pltpu.async_copy(
                table_hbm.at[idx_v.at[c * CHUNK_BR + j]],
                rows[b].at[j],
                gsem[b],
            )

    def wait_gathers(c, b):
        for j in range(CHUNK_BR):
            pltpu.make_async_copy(
                table_hbm.at[idx_v.at[c * CHUNK_BR + j]],
                rows[b].at[j],
                gsem[b],
            ).wait()

    def fire_store(c, b):
        pltpu.async_copy(
            rows[b],
            out_hbm.at[pl.ds(br_base + c * CHUNK_BR, CHUNK_BR), :, pl.ds(0, DIM)],
            ssem[b],
        )

    def wait_store(b):
        pltpu.make_async_copy(
            rows[b],
            out_hbm.at[pl.ds(0, CHUNK_BR), :, pl.ds(0, DIM)],
            ssem[b],
        ).wait()

    fire_gathers(0, 0)
    fire_gathers(1, 1)
    wait_gathers(0, 0)
    fire_store(0, 0)
    fire_gathers(2, 2)
    wait_gathers(1, 1)
    fire_store(1, 1)

    def group(g, carry):
        for k in range(NBUF):
            c = NBUF + g * NBUF + k
            prev = (k + NBUF - 1) % NBUF
            wait_store(k)
            fire_gathers(c, k)
            wait_gathers(c - 1, prev)
            fire_store(c - 1, prev)
        return carry

    ngroups = (NCHUNK - NBUF) // NBUF
    lax.fori_loop(0, ngroups, group, None)

    for c in range(NBUF + ngroups * NBUF, NCHUNK):
        b = c % NBUF
        prev = (b + NBUF - 1) % NBUF
        wait_store(b)
        fire_gathers(c, b)
        wait_gathers(c - 1, prev)
        fire_store(c - 1, prev)

    last = (NCHUNK - 1) % NBUF
    wait_gathers(NCHUNK - 1, last)
    fire_store(NCHUNK - 1, last)
    for b in range(NBUF):
        wait_store(b)


def kernel(x, embedding):
    x2 = jnp.pad(x.astype(jnp.int32), ((0, 0), (0, SP - S)))
    mesh = plsc.VectorSubcoreMesh(core_axis_name="c", subcore_axis_name="s")
    out = pl.kernel(
        _gather_body,
        mesh=mesh,
        compiler_params=pltpu.CompilerParams(use_tc_tiling_on_sc=False),
        out_type=jax.ShapeDtypeStruct((B, SP, DP), jnp.float32),
        scratch_types=[
            pltpu.VMEM((BR_PER_W, SP), jnp.int32),
            pltpu.VMEM((CHUNK_BR, SP, DIM), jnp.float32),
            pltpu.VMEM((CHUNK_BR, SP, DIM), jnp.float32),
            pltpu.VMEM((CHUNK_BR, SP, DIM), jnp.float32),
            pltpu.SemaphoreType.DMA,
            pltpu.SemaphoreType.DMA,
            pltpu.SemaphoreType.DMA,
            pltpu.SemaphoreType.DMA,
            pltpu.SemaphoreType.DMA,
            pltpu.SemaphoreType.DMA,
        ],
    )(x2, embedding)
    return out[:, :S, :DIM]


# speedup vs baseline: 1.8468x; 1.8468x over previous
import jax
import jax.numpy as jnp
from jax import lax
from jax.experimental import pallas as pl
from jax.experimental.pallas import tpu as pltpu
from jax.experimental.pallas import tpu_sc as plsc

B = 16384
S = 50
DIM = 32
NC = 2
NS = 16
NW = NC * NS
BR_PER_W = B // NW
CHUNK_BR = 8
NCHUNK = BR_PER_W // CHUNK_BR
NBUF = 3


def _gather_body(x_hbm, table_hbm, out_hbm,
                 idx_v, rows0, rows1, rows2,
                 gsem0, gsem1, gsem2, ssem0, ssem1, ssem2):
    rows = (rows0, rows1, rows2)
    gsem = (gsem0, gsem1, gsem2)
    ssem = (ssem0, ssem1, ssem2)

    wid = lax.axis_index("s") * NC + lax.axis_index("c")
    br_base = wid * BR_PER_W

    pltpu.sync_copy(x_hbm.at[pl.ds(br_base, BR_PER_W)], idx_v)

    def fire_gathers(c, b):
        for j in range(CHUNK_BR):
            pltpu.async_copy(
                table_hbm.at[idx_v.at[c * CHUNK_BR + j]],
                rows[b].at[j],
                gsem[b],
            )

    def wait_gathers(b):
        pltpu.make_async_copy(
            out_hbm.at[pl.ds(0, CHUNK_BR)], rows[b], gsem[b]
        ).wait()

    def fire_store(c, b):
        pltpu.async_copy(
            rows[b],
            out_hbm.at[pl.ds(br_base + c * CHUNK_BR, CHUNK_BR)],
            ssem[b],
        )

    def wait_store(b):
        pltpu.make_async_copy(
            rows[b], out_hbm.at[pl.ds(0, CHUNK_BR)], ssem[b]
        ).wait()

    fire_gathers(0, 0)
    fire_gathers(1, 1)
    wait_gathers(0)
    fire_store(0, 0)
    fire_gathers(2, 2)
    wait_gathers(1)
    fire_store(1, 1)

    def group(g, carry):
        for k in range(NBUF):
            c = NBUF + g * NBUF + k
            prev = (k + NBUF - 1) % NBUF
            wait_store(k)
            fire_gathers(c, k)
            wait_gathers(prev)
            fire_store(c - 1, prev)
        return carry

    ngroups = (NCHUNK - NBUF) // NBUF
    lax.fori_loop(0, ngroups, group, None)

    for c in range(NBUF + ngroups * NBUF, NCHUNK):
        b = c % NBUF
        prev = (b + NBUF - 1) % NBUF
        wait_store(b)
        fire_gathers(c, b)
        wait_gathers(prev)
        fire_store(c - 1, prev)

    last = (NCHUNK - 1) % NBUF
    wait_gathers(last)
    fire_store(NCHUNK - 1, last)
    for b in range(NBUF):
        wait_store(b)


def kernel(x, embedding):
    xi = x.astype(jnp.int32)
    mesh = plsc.VectorSubcoreMesh(core_axis_name="c", subcore_axis_name="s")
    out = pl.kernel(
        _gather_body,
        mesh=mesh,
        compiler_params=pltpu.CompilerParams(use_tc_tiling_on_sc=False),
        out_type=jax.ShapeDtypeStruct((B, S, DIM), jnp.float32),
        scratch_types=[
            pltpu.VMEM((BR_PER_W, S), jnp.int32),
            pltpu.VMEM((CHUNK_BR, S, DIM), jnp.float32),
            pltpu.VMEM((CHUNK_BR, S, DIM), jnp.float32),
            pltpu.VMEM((CHUNK_BR, S, DIM), jnp.float32),
            pltpu.SemaphoreType.DMA,
            pltpu.SemaphoreType.DMA,
            pltpu.SemaphoreType.DMA,
            pltpu.SemaphoreType.DMA,
            pltpu.SemaphoreType.DMA,
            pltpu.SemaphoreType.DMA,
        ],
    )(xi, embedding)
    return out
